# Initial kernel scaffold; baseline (speedup 1.0000x reference)
import jax
import jax.numpy as jnp
from jax.experimental import pallas as pl


def kernel(x, edge_index, batch, params):
    raise NotImplementedError("write your pallas kernel here")


# speedup vs baseline: 3.6633x; 3.6633x over previous
import functools

import jax
import jax.numpy as jnp
from jax import lax
from jax.experimental import pallas as pl
from jax.experimental.pallas import tpu as pltpu
from jax.experimental.pallas import tpu_sc as plsc

HID = 64
NG = 64
HALF = HID // 2


@functools.cache
def _make_edge_segsum(N, E):
    NC, NS, CH = 2, 16, 128
    nchunk = E // CH
    n_iter = (nchunk + NS - 1) // NS
    zrows = 400
    nzchunk = N // zrows
    nz_iter = (nzchunk + NS - 1) // NS

    mesh = plsc.VectorSubcoreMesh(core_axis_name="c", subcore_axis_name="s")

    @functools.partial(
        pl.kernel,
        out_type=jax.ShapeDtypeStruct((NC * N, HALF), jnp.float32),
        mesh=mesh,
        scratch_types=[
            pltpu.VMEM((CH,), jnp.int32),
            pltpu.VMEM((CH,), jnp.int32),
            pltpu.VMEM((CH,), jnp.int32),
            pltpu.VMEM((CH, HALF), jnp.float32),
            pltpu.VMEM((zrows, HALF), jnp.float32),
            pltpu.VMEM_SHARED((N, HALF), jnp.float32),
            pltpu.SemaphoreType.DMA,
        ],
        compiler_params=pltpu.CompilerParams(use_tc_tiling_on_sc=False),
    )
    def seg(h2, srcr, dstr, out, srcv, dstv, idxv, rows, zbuf, acc, sem):
        c = lax.axis_index("c")
        s = lax.axis_index("s")
        zero16 = jnp.zeros((16,), jnp.float32)

        def zb(i, carry):
            zbuf[i, pl.ds(0, 16)] = zero16
            zbuf[i, pl.ds(16, 16)] = zero16
            return carry

        lax.fori_loop(0, zrows, zb, 0)

        def zr(k, carry):
            zc = s + k * NS

            @pl.when(zc < nzchunk)
            def _():
                pltpu.sync_copy(zbuf, acc.at[pl.ds(zc * zrows, zrows)])

            return carry

        lax.fori_loop(0, nz_iter, zr, 0)
        plsc.subcore_barrier()

        def body(k, carry):
            chunk = s + k * NS

            @pl.when(chunk < nchunk)
            def _():
                base = chunk * CH
                pltpu.sync_copy(srcr.at[pl.ds(base, CH)], srcv)
                pltpu.sync_copy(dstr.at[pl.ds(base, CH)], dstv)
                for i in range(CH // 16):
                    sl = pl.ds(i * 16, 16)
                    idxv[sl] = srcv[sl] * 2 + c
                pltpu.async_copy(h2.at[idxv], rows, sem).wait()
                pltpu.sync_copy(rows, acc.at[dstv], add=True)

            return carry

        lax.fori_loop(0, n_iter, body, 0)
        plsc.subcore_barrier()

        def wb(k, carry):
            zc = s + k * NS

            @pl.when(zc < nzchunk)
            def _():
                pltpu.sync_copy(acc.at[pl.ds(zc * zrows, zrows)],
                                out.at[pl.ds(c * N + zc * zrows, zrows)])

            return carry

        lax.fori_loop(0, nz_iter, wb, 0)

    return seg


def _pool_contrib(batch_ref, hn, bn):
    bt = batch_ref[0]
    ohT = (lax.broadcasted_iota(jnp.int32, (NG, bn), 0) == bt).astype(jnp.float32)
    return jnp.dot(ohT, hn, preferred_element_type=jnp.float32)


@functools.cache
def _make_pre(N, F, bn):
    G = N // bn

    def body(x_ref, batch_ref, w_ref, b_ref, h_ref, pool_ref):
        i = pl.program_id(0)
        h = jnp.dot(x_ref[...], w_ref[...], preferred_element_type=jnp.float32)
        h = h + b_ref[...]
        h_ref[...] = h
        contrib = _pool_contrib(batch_ref, h, bn)

        @pl.when(i == 0)
        def _():
            pool_ref[...] = contrib

        @pl.when(i != 0)
        def _():
            pool_ref[...] = pool_ref[...] + contrib

    return pl.pallas_call(
        body,
        grid=(G,),
        in_specs=[
            pl.BlockSpec((bn, F), lambda i: (i, 0)),
            pl.BlockSpec((1, 1, bn), lambda i: (i, 0, 0)),
            pl.BlockSpec((F, HID), lambda i: (0, 0)),
            pl.BlockSpec((1, HID), lambda i: (0, 0)),
        ],
        out_specs=[
            pl.BlockSpec((bn, HID), lambda i: (i, 0)),
            pl.BlockSpec((NG, HID), lambda i: (0, 0)),
        ],
        out_shape=[
            jax.ShapeDtypeStruct((N, HID), jnp.float32),
            jax.ShapeDtypeStruct((NG, HID), jnp.float32),
        ],
    )


@functools.cache
def _make_mlp(N, bn, residual):
    G = N // bn

    def body(*refs):
        if residual:
            (h_ref, agg_ref, hres_ref, batch_ref, w1_ref, b1_ref, w2_ref,
             b2_ref, hout_ref, hresout_ref, pool_ref) = refs
        else:
            (h_ref, agg_ref, batch_ref, w1_ref, b1_ref, w2_ref, b2_ref,
             hout_ref, pool_ref) = refs
        i = pl.program_id(0)
        a = jnp.concatenate([agg_ref[0], agg_ref[1]], axis=1)
        z = h_ref[...] + a
        z = jnp.dot(z, w1_ref[...], preferred_element_type=jnp.float32) + b1_ref[...]
        z = jnp.maximum(z, 0.0)
        z = jnp.dot(z, w2_ref[...], preferred_element_type=jnp.float32) + b2_ref[...]
        if residual:
            z = z + hres_ref[...]
            hresout_ref[...] = z
        hn = jnp.maximum(z, 0.0)
        hout_ref[...] = hn
        contrib = _pool_contrib(batch_ref, hn, bn)

        @pl.when(i == 0)
        def _():
            pool_ref[...] = contrib

        @pl.when(i != 0)
        def _():
            pool_ref[...] = pool_ref[...] + contrib

    in_specs = [
        pl.BlockSpec((bn, HID), lambda i: (i, 0)),
        pl.BlockSpec((2, bn, HALF), lambda i: (0, i, 0)),
    ]
    if residual:
        in_specs.append(pl.BlockSpec((bn, HID), lambda i: (i, 0)))
    in_specs += [
        pl.BlockSpec((1, 1, bn), lambda i: (i, 0, 0)),
        pl.BlockSpec((HID, HID), lambda i: (0, 0)),
        pl.BlockSpec((1, HID), lambda i: (0, 0)),
        pl.BlockSpec((HID, HID), lambda i: (0, 0)),
        pl.BlockSpec((1, HID), lambda i: (0, 0)),
    ]
    out_specs = [pl.BlockSpec((bn, HID), lambda i: (i, 0))]
    out_shape = [jax.ShapeDtypeStruct((N, HID), jnp.float32)]
    if residual:
        out_specs.append(pl.BlockSpec((bn, HID), lambda i: (i, 0)))
        out_shape.append(jax.ShapeDtypeStruct((N, HID), jnp.float32))
    out_specs.append(pl.BlockSpec((NG, HID), lambda i: (0, 0)))
    out_shape.append(jax.ShapeDtypeStruct((NG, HID), jnp.float32))

    return pl.pallas_call(
        body,
        grid=(G,),
        in_specs=in_specs,
        out_specs=out_specs,
        out_shape=out_shape,
    )


@functools.cache
def _make_post(OUT, CAT):
    def body(p_ref, w1_ref, b1_ref, w2_ref, b2_ref, out_ref):
        p = p_ref[...]
        z = jnp.dot(p, w1_ref[...], preferred_element_type=jnp.float32) + b1_ref[...]
        z = jnp.maximum(z, 0.0)
        out_ref[...] = (
            jnp.dot(z, w2_ref[...], preferred_element_type=jnp.float32) + b2_ref[...]
        )

    return pl.pallas_call(
        body,
        out_shape=jax.ShapeDtypeStruct((NG, OUT), jnp.float32),
    )


def kernel(x, edge_index, batch, params):
    N, F = x.shape
    E = edge_index.shape[1]
    OUT = params["post_W2"].shape[1]
    src = edge_index[0]
    dst = edge_index[1]
    bn = 1000
    G = N // bn
    batch3 = batch.reshape(G, 1, bn)

    pre = _make_pre(N, F, bn)
    h, p0 = pre(x, batch3, params["pre_W"], params["pre_b"].reshape(1, HID))

    segsum = _make_edge_segsum(N, E)
    mlp_plain = _make_mlp(N, bn, False)
    mlp_res = _make_mlp(N, bn, True)

    pools = [p0]
    hres = h
    for l in range(4):
        agg2 = segsum(h.reshape(2 * N, HALF), src, dst).reshape(2, N, HALF)
        w1 = params["conv%d_W1" % l]
        b1 = params["conv%d_b1" % l].reshape(1, HID)
        w2 = params["conv%d_W2" % l]
        b2 = params["conv%d_b2" % l].reshape(1, HID)
        if l & 1:
            h, hres, pc = mlp_res(h, agg2, hres, batch3, w1, b1, w2, b2)
        else:
            h, pc = mlp_plain(h, agg2, batch3, w1, b1, w2, b2)
        pools.append(pc)

    pooled = jnp.concatenate(pools, axis=1)
    post = _make_post(OUT, pooled.shape[1])
    return post(
        pooled,
        params["post_W1"],
        params["post_b1"].reshape(1, HID),
        params["post_W2"],
        params["post_b2"].reshape(1, OUT),
    )


# speedup vs baseline: 9.5718x; 2.6129x over previous
import functools

import jax
import jax.numpy as jnp
from jax import lax
from jax.experimental import pallas as pl
from jax.experimental.pallas import tpu as pltpu
from jax.experimental.pallas import tpu_sc as plsc

HID = 64
NG = 64
HALF = HID // 2


@functools.cache
def _make_edge_segsum(N, E):
    NC, NS, CH = 2, 16, 128
    NB = 4
    SLK = 32
    nchunk = E // CH
    full_slabs = (nchunk // SLK) // NS * NS
    slabs_per_tile = full_slabs // NS
    groups_per_slab = SLK // NB
    tail0 = full_slabs * SLK
    tail_chunks = nchunk - tail0
    tail_iter = (tail_chunks + NS - 1) // NS
    zrows = 80
    nzchunk = N // zrows
    nz_iter = (nzchunk + NS - 1) // NS

    mesh = plsc.VectorSubcoreMesh(core_axis_name="c", subcore_axis_name="s")

    @functools.partial(
        pl.kernel,
        out_type=jax.ShapeDtypeStruct((NC * N, HALF), jnp.float32),
        mesh=mesh,
        scratch_types=[
            pltpu.VMEM((SLK, CH), jnp.int32),
            pltpu.VMEM((SLK, CH), jnp.int32),
            pltpu.VMEM((NB, CH), jnp.int32),
            pltpu.VMEM((NB, CH, HALF), jnp.float32),
            pltpu.VMEM((zrows, HALF), jnp.float32),
            pltpu.VMEM_SHARED((N, HALF), jnp.float32),
        ] + [pltpu.SemaphoreType.DMA] * (2 * NB),
        compiler_params=pltpu.CompilerParams(use_tc_tiling_on_sc=False),
    )
    def seg(h2, src2d, dst2d, out, srcsl, dstsl, idxb, rows, zbuf, acc, *sems):
        gsems = sems[:NB]
        ssems = sems[NB:]
        c = lax.axis_index("c")
        s = lax.axis_index("s")
        zero16 = jnp.zeros((16,), jnp.float32)

        def zb(i, carry):
            zbuf[i, pl.ds(0, 16)] = zero16
            zbuf[i, pl.ds(16, 16)] = zero16
            return carry

        lax.fori_loop(0, zrows, zb, 0)

        def zr(k, carry):
            zc = s + k * NS

            @pl.when(zc < nzchunk)
            def _():
                pltpu.sync_copy(zbuf, acc.at[pl.ds(zc * zrows, zrows)])

            return carry

        lax.fori_loop(0, nz_iter, zr, 0)
        plsc.subcore_barrier()

        def slab_body(sl_i, carry):
            slab = s * slabs_per_tile + sl_i
            base_chunk = slab * SLK
            pltpu.sync_copy(src2d.at[pl.ds(base_chunk, SLK)], srcsl)
            pltpu.sync_copy(dst2d.at[pl.ds(base_chunk, SLK)], dstsl)

            def group(gi, gcarry):
                base = gi * NB
                for b in range(NB):
                    j = base + b

                    @pl.when(gi > 0)
                    def _():
                        pltpu.make_async_copy(
                            rows.at[b], acc.at[dstsl.at[0]], ssems[b]
                        ).wait()

                    for i in range(CH // 16):
                        slc = pl.ds(i * 16, 16)
                        idxb[b, slc] = srcsl[j, slc] * 2 + c
                    pltpu.async_copy(h2.at[idxb.at[b]], rows.at[b], gsems[b])
                for b in range(NB):
                    j = base + b
                    pltpu.make_async_copy(
                        h2.at[idxb.at[b]], rows.at[b], gsems[b]
                    ).wait()
                    pltpu.async_copy(
                        rows.at[b], acc.at[dstsl.at[j]], ssems[b], add=True
                    )
                return gcarry

            lax.fori_loop(0, groups_per_slab, group, 0)
            for b in range(NB):
                pltpu.make_async_copy(
                    rows.at[b], acc.at[dstsl.at[0]], ssems[b]
                ).wait()
            return carry

        lax.fori_loop(0, slabs_per_tile, slab_body, 0)

        if tail_chunks:
            def tail(k, carry):
                t = tail0 + s + k * NS

                @pl.when(t < nchunk)
                def _():
                    pltpu.sync_copy(src2d.at[t], srcsl.at[0])
                    pltpu.sync_copy(dst2d.at[t], dstsl.at[0])
                    for i in range(CH // 16):
                        slc = pl.ds(i * 16, 16)
                        idxb[0, slc] = srcsl[0, slc] * 2 + c
                    pltpu.async_copy(
                        h2.at[idxb.at[0]], rows.at[0], gsems[0]
                    ).wait()
                    pltpu.sync_copy(rows.at[0], acc.at[dstsl.at[0]], add=True)

                return carry

            lax.fori_loop(0, tail_iter, tail, 0)

        plsc.subcore_barrier()

        def wb(k, carry):
            zc = s + k * NS

            @pl.when(zc < nzchunk)
            def _():
                pltpu.sync_copy(acc.at[pl.ds(zc * zrows, zrows)],
                                out.at[pl.ds(c * N + zc * zrows, zrows)])

            return carry

        lax.fori_loop(0, nz_iter, wb, 0)

    return seg


def _pool_contrib(batch_ref, hn, bn):
    bt = batch_ref[0]
    ohT = (lax.broadcasted_iota(jnp.int32, (NG, bn), 0) == bt).astype(jnp.float32)
    return jnp.dot(ohT, hn, preferred_element_type=jnp.float32)


@functools.cache
def _make_pre(N, F, bn):
    G = N // bn

    def body(x_ref, batch_ref, w_ref, b_ref, h_ref, pool_ref):
        i = pl.program_id(0)
        h = jnp.dot(x_ref[...], w_ref[...], preferred_element_type=jnp.float32)
        h = h + b_ref[...]
        h_ref[...] = h
        contrib = _pool_contrib(batch_ref, h, bn)

        @pl.when(i == 0)
        def _():
            pool_ref[...] = contrib

        @pl.when(i != 0)
        def _():
            pool_ref[...] = pool_ref[...] + contrib

    return pl.pallas_call(
        body,
        grid=(G,),
        in_specs=[
            pl.BlockSpec((bn, F), lambda i: (i, 0)),
            pl.BlockSpec((1, 1, bn), lambda i: (i, 0, 0)),
            pl.BlockSpec((F, HID), lambda i: (0, 0)),
            pl.BlockSpec((1, HID), lambda i: (0, 0)),
        ],
        out_specs=[
            pl.BlockSpec((bn, HID), lambda i: (i, 0)),
            pl.BlockSpec((NG, HID), lambda i: (0, 0)),
        ],
        out_shape=[
            jax.ShapeDtypeStruct((N, HID), jnp.float32),
            jax.ShapeDtypeStruct((NG, HID), jnp.float32),
        ],
    )


@functools.cache
def _make_mlp(N, bn, residual):
    G = N // bn

    def body(*refs):
        if residual:
            (h_ref, agg_ref, hres_ref, batch_ref, w1_ref, b1_ref, w2_ref,
             b2_ref, hout_ref, hresout_ref, pool_ref) = refs
        else:
            (h_ref, agg_ref, batch_ref, w1_ref, b1_ref, w2_ref, b2_ref,
             hout_ref, pool_ref) = refs
        i = pl.program_id(0)
        a = jnp.concatenate([agg_ref[0], agg_ref[1]], axis=1)
        z = h_ref[...] + a
        z = jnp.dot(z, w1_ref[...], preferred_element_type=jnp.float32) + b1_ref[...]
        z = jnp.maximum(z, 0.0)
        z = jnp.dot(z, w2_ref[...], preferred_element_type=jnp.float32) + b2_ref[...]
        if residual:
            z = z + hres_ref[...]
            hresout_ref[...] = z
        hn = jnp.maximum(z, 0.0)
        hout_ref[...] = hn
        contrib = _pool_contrib(batch_ref, hn, bn)

        @pl.when(i == 0)
        def _():
            pool_ref[...] = contrib

        @pl.when(i != 0)
        def _():
            pool_ref[...] = pool_ref[...] + contrib

    in_specs = [
        pl.BlockSpec((bn, HID), lambda i: (i, 0)),
        pl.BlockSpec((2, bn, HALF), lambda i: (0, i, 0)),
    ]
    if residual:
        in_specs.append(pl.BlockSpec((bn, HID), lambda i: (i, 0)))
    in_specs += [
        pl.BlockSpec((1, 1, bn), lambda i: (i, 0, 0)),
        pl.BlockSpec((HID, HID), lambda i: (0, 0)),
        pl.BlockSpec((1, HID), lambda i: (0, 0)),
        pl.BlockSpec((HID, HID), lambda i: (0, 0)),
        pl.BlockSpec((1, HID), lambda i: (0, 0)),
    ]
    out_specs = [pl.BlockSpec((bn, HID), lambda i: (i, 0))]
    out_shape = [jax.ShapeDtypeStruct((N, HID), jnp.float32)]
    if residual:
        out_specs.append(pl.BlockSpec((bn, HID), lambda i: (i, 0)))
        out_shape.append(jax.ShapeDtypeStruct((N, HID), jnp.float32))
    out_specs.append(pl.BlockSpec((NG, HID), lambda i: (0, 0)))
    out_shape.append(jax.ShapeDtypeStruct((NG, HID), jnp.float32))

    return pl.pallas_call(
        body,
        grid=(G,),
        in_specs=in_specs,
        out_specs=out_specs,
        out_shape=out_shape,
    )


@functools.cache
def _make_post(OUT, CAT):
    def body(p_ref, w1_ref, b1_ref, w2_ref, b2_ref, out_ref):
        p = p_ref[...]
        z = jnp.dot(p, w1_ref[...], preferred_element_type=jnp.float32) + b1_ref[...]
        z = jnp.maximum(z, 0.0)
        out_ref[...] = (
            jnp.dot(z, w2_ref[...], preferred_element_type=jnp.float32) + b2_ref[...]
        )

    return pl.pallas_call(
        body,
        out_shape=jax.ShapeDtypeStruct((NG, OUT), jnp.float32),
    )


def kernel(x, edge_index, batch, params):
    N, F = x.shape
    E = edge_index.shape[1]
    OUT = params["post_W2"].shape[1]
    src = edge_index[0]
    dst = edge_index[1]
    bn = 1000
    G = N // bn
    batch3 = batch.reshape(G, 1, bn)

    pre = _make_pre(N, F, bn)
    h, p0 = pre(x, batch3, params["pre_W"], params["pre_b"].reshape(1, HID))

    segsum = _make_edge_segsum(N, E)
    mlp_plain = _make_mlp(N, bn, False)
    mlp_res = _make_mlp(N, bn, True)

    src2d = src.reshape(E // 128, 128)
    dst2d = dst.reshape(E // 128, 128)
    pools = [p0]
    hres = h
    for l in range(4):
        agg2 = segsum(h.reshape(2 * N, HALF), src2d, dst2d).reshape(2, N, HALF)
        w1 = params["conv%d_W1" % l]
        b1 = params["conv%d_b1" % l].reshape(1, HID)
        w2 = params["conv%d_W2" % l]
        b2 = params["conv%d_b2" % l].reshape(1, HID)
        if l & 1:
            h, hres, pc = mlp_res(h, agg2, hres, batch3, w1, b1, w2, b2)
        else:
            h, pc = mlp_plain(h, agg2, batch3, w1, b1, w2, b2)
        pools.append(pc)

    pooled = jnp.concatenate(pools, axis=1)
    post = _make_post(OUT, pooled.shape[1])
    return post(
        pooled,
        params["post_W1"],
        params["post_b1"].reshape(1, HID),
        params["post_W2"],
        params["post_b2"].reshape(1, OUT),
    )
